# baseline (speedup 1.0000x reference)
import jax
import jax.numpy as jnp
from jax import lax
from jax.experimental import pallas as pl
from jax.experimental.pallas import tpu as pltpu
from jax.experimental.pallas import tpu_sc as plsc

N = 10000
E = 160000
D_IN = 256
D_HID = 512

NC = 2
NS = 16
SLICE = 128
DEGW = 16
NP = 10240
K = 80
NCHUNK = 128
NBUF = 4
GB = 16
NGRP = NCHUNK // GB
NHALF = NGRP // 2
EPT = NCHUNK * K
EPAD = NS * EPT
RPT = NP // NS
ZR = K
NZ = RPT // ZR

_MESH = plsc.VectorSubcoreMesh(core_axis_name="c", subcore_axis_name="s")

def _make_seg_kernel(n_slices, compute_deg):
    n_passes = n_slices // NC
    out_type = [jax.ShapeDtypeStruct((n_slices, NP, SLICE), jnp.float32)]
    if compute_deg:
        out_type.append(jax.ShapeDtypeStruct((NC, NP, SLICE), jnp.float32))

    scratch = [
        pltpu.VMEM_SHARED((NP, SLICE), jnp.float32),
        pltpu.VMEM((2, GB, K), jnp.int32),
        pltpu.VMEM((2, GB, K), jnp.int32),
    ] + [pltpu.VMEM((K, SLICE), jnp.float32) for _ in range(NBUF)] \
      + [pltpu.SemaphoreType.DMA for _ in range(2 * NBUF + 2)]

    def body(*refs):
        n_fixed = 5 if compute_deg else 4
        if compute_deg:
            (a_all, src_hbm, dst_hbm, o_all, deg_hbm) = refs[:5]
        else:
            (a_all, src_hbm, dst_hbm, o_all) = refs[:4]
        acc, sidx2, didx2 = refs[n_fixed:n_fixed + 3]
        bufs = refs[n_fixed + 3:n_fixed + 3 + NBUF]
        sems = refs[n_fixed + 3 + NBUF:]
        gsems = sems[:NBUF]
        ssems = sems[NBUF:2 * NBUF]
        isems = sems[2 * NBUF:2 * NBUF + 2]
        rows = bufs[0]

        c = lax.axis_index("c")
        s = lax.axis_index("s")
        base = s * RPT

        def fill_rows(val):
            def zb(i, carry):
                for q in range(SLICE // 16):
                    rows[i, pl.ds(q * 16, 16)] = jnp.full((16,), val,
                                                          jnp.float32)
                return carry
            lax.fori_loop(0, ZR, zb, 0)

        def zero_acc():
            fill_rows(0.0)
            for q in range(NZ):
                pltpu.sync_copy(rows, acc.at[pl.ds(base + q * ZR, ZR)])

        def write_acc(o_ref):
            pend = [None, None]
            for q in range(NZ):
                b = q % 2
                if pend[b] is not None:
                    pend[b].wait()
                pltpu.sync_copy(acc.at[pl.ds(base + q * ZR, ZR)], bufs[b])
                pend[b] = pltpu.async_copy(
                    bufs[b], o_ref.at[pl.ds(base + q * ZR, ZR)], gsems[b])
            for b in range(2):
                if pend[b] is not None:
                    pend[b].wait()

        def stage_idx(g, par, sync):
            dsts = (sidx2.at[par], didx2.at[par])
            srcs = (src_hbm, dst_hbm)
            if sync:
                for sr, dr in zip(srcs, dsts):
                    pltpu.sync_copy(sr.at[s].at[pl.ds(g * GB, GB)], dr)
                return None
            return [pltpu.async_copy(sr.at[s].at[pl.ds(g * GB, GB)], dr,
                                     isems[i])
                    for i, (sr, dr) in enumerate(zip(srcs, dsts))]

        def run_group(a_ref, sidx_g, didx_g):
            pend_g = [
                pltpu.async_copy(a_ref.at[sidx_g.at[tt]], bufs[tt], gsems[tt])
                for tt in range(NBUF)
            ]
            pend_s = [None] * NBUF
            for t in range(GB):
                b = t % NBUF
                if t >= 1 and (t - 1) + NBUF < GB:
                    bp = (t - 1) % NBUF
                    pend_s[bp].wait()
                    pend_s[bp] = None
                    pend_g[bp] = pltpu.async_copy(
                        a_ref.at[sidx_g.at[t - 1 + NBUF]], bufs[bp],
                        gsems[bp])
                pend_g[b].wait()
                pend_s[b] = pltpu.async_copy(
                    bufs[b], acc.at[didx_g.at[t]], ssems[b], add=True)
            for b in range(NBUF):
                if pend_s[b] is not None:
                    pend_s[b].wait()

        for p in range(n_passes):
            j = c * n_passes + p
            a_ref = a_all.at[j]
            o_ref = o_all.at[j]

            zero_acc()
            plsc.subcore_barrier()

            stage_idx(0, 0, True)

            def pair(u, carry):
                g0 = 2 * u
                pf = stage_idx(g0 + 1, 1, False)
                run_group(a_ref, sidx2.at[0], didx2.at[0])
                for d in pf:
                    d.wait()
                gnext = jnp.minimum(g0 + 2, NGRP - 2)
                pf2 = stage_idx(gnext, 0, False)
                run_group(a_ref, sidx2.at[1], didx2.at[1])
                for d in pf2:
                    d.wait()
                return carry
            lax.fori_loop(0, NHALF, pair, 0)
            plsc.subcore_barrier()

            write_acc(o_ref)
            plsc.subcore_barrier()

        if compute_deg:
            zero_acc()
            fill_rows(1.0)
            plsc.subcore_barrier()

            def dgroup(u, carry):
                g = c * NHALF + u
                stage_idx(g, 0, True)
                pend = [None] * NBUF
                for tt in range(GB):
                    b = tt % NBUF
                    if pend[b] is not None:
                        pend[b].wait()
                    pend[b] = pltpu.async_copy(
                        rows, acc.at[didx2.at[0].at[tt]], ssems[b], add=True)
                for b in range(NBUF):
                    if pend[b] is not None:
                        pend[b].wait()
                return carry
            lax.fori_loop(0, NHALF, dgroup, 0)
            plsc.subcore_barrier()

            write_acc(deg_hbm.at[c])

    return pl.kernel(body, out_type=tuple(out_type), mesh=_MESH,
                     scratch_types=scratch)


_seg2_deg = _make_seg_kernel(2, True)
_seg4 = _make_seg_kernel(4, False)


def _make_tc_layer(n_in, relu, sliced_out):
    d_in = n_in * SLICE
    RB = 1024
    grid = (NP // RB,)
    n_out = D_HID // SLICE

    def body(a_ref, h_ref, deg_ref, wl_ref, wr_ref, out_ref):
        accl = jnp.zeros((RB, D_HID), jnp.float32)
        accr = jnp.zeros((RB, D_HID), jnp.float32)
        for j in range(n_in):
            accl += jnp.dot(a_ref[j], wl_ref[j * SLICE:(j + 1) * SLICE, :],
                            preferred_element_type=jnp.float32)
            accr += jnp.dot(h_ref[j], wr_ref[j * SLICE:(j + 1) * SLICE, :],
                            preferred_element_type=jnp.float32)
        invd = 1.0 / jnp.maximum(deg_ref[0][:, 0:1] + deg_ref[1][:, 0:1],
                                 1.0)
        res = accl * invd + accr
        if relu:
            res = jnp.maximum(res, 0.0)
        if sliced_out:
            for j in range(n_out):
                out_ref[j] = res[:, j * SLICE:(j + 1) * SLICE]
        else:
            out_ref[...] = res

    in_specs = [
        pl.BlockSpec((n_in, RB, SLICE), lambda i: (0, i, 0)),
        pl.BlockSpec((n_in, RB, SLICE), lambda i: (0, i, 0)),
        pl.BlockSpec((NC, RB, SLICE), lambda i: (0, i, 0)),
        pl.BlockSpec((d_in, D_HID), lambda i: (0, 0)),
        pl.BlockSpec((d_in, D_HID), lambda i: (0, 0)),
    ]
    if sliced_out:
        out_specs = pl.BlockSpec((n_out, RB, SLICE), lambda i: (0, i, 0))
        out_shape = jax.ShapeDtypeStruct((n_out, NP, SLICE), jnp.float32)
    else:
        out_specs = pl.BlockSpec((RB, D_HID), lambda i: (i, 0))
        out_shape = jax.ShapeDtypeStruct((NP, D_HID), jnp.float32)

    return pl.pallas_call(body, grid=grid, in_specs=in_specs,
                          out_specs=out_specs, out_shape=out_shape)


_tc1 = _make_tc_layer(D_IN // SLICE, True, True)
_tc2 = _make_tc_layer(D_HID // SLICE, True, True)
_tc3 = _make_tc_layer(D_HID // SLICE, False, False)


def kernel(x, edge_index, W1_l, W1_r, W2_l, W2_r, W3_l, W3_r):
    e32 = edge_index.astype(jnp.int32)
    npad = EPAD - E
    src = jnp.concatenate([e32[0], jnp.zeros((npad,), jnp.int32)])
    dst = jnp.concatenate(
        [e32[1], N + (jnp.arange(npad, dtype=jnp.int32) % (NP - N))])
    src = src.reshape(NS, NCHUNK, K)
    dst = dst.reshape(NS, NCHUNK, K)

    x_all = jnp.pad(x.reshape(N, 2, SLICE).transpose(1, 0, 2),
                    ((0, 0), (0, NP - N), (0, 0)))

    a_all, deg = _seg2_deg(x_all, src, dst)
    h1 = _tc1(a_all, x_all, deg, W1_l, W1_r)

    (b_all,) = _seg4(h1, src, dst)
    h2 = _tc2(b_all, h1, deg, W2_l, W2_r)

    (c_all,) = _seg4(h2, src, dst)
    out = _tc3(c_all, h2, deg, W3_l, W3_r)
    return out[:N]


# speedup vs baseline: 1.0160x; 1.0160x over previous
import jax
import jax.numpy as jnp
from jax import lax
from jax.experimental import pallas as pl
from jax.experimental.pallas import tpu as pltpu
from jax.experimental.pallas import tpu_sc as plsc

N = 10000
E = 160000
D_IN = 256
D_HID = 512

NC = 2
NS = 16
SLICE = 128
DEGW = 16
NP = 10240
K = 80
NCHUNK = 128
NBUF = 3
GB = 32
NGRP = NCHUNK // GB
NHALF = NGRP // 2
EPT = NCHUNK * K
EPAD = NS * EPT
RPT = NP // NS
ZR = K
NZ = RPT // ZR

_MESH = plsc.VectorSubcoreMesh(core_axis_name="c", subcore_axis_name="s")

def _make_seg_kernel(n_slices, compute_deg):
    n_passes = n_slices // NC
    out_type = [jax.ShapeDtypeStruct((n_slices, NP, SLICE), jnp.float32)]
    if compute_deg:
        out_type.append(jax.ShapeDtypeStruct((NC, NP, SLICE), jnp.float32))

    scratch = [
        pltpu.VMEM_SHARED((NP, SLICE), jnp.float32),
        pltpu.VMEM((2, GB, K), jnp.int32),
        pltpu.VMEM((2, GB, K), jnp.int32),
    ] + [pltpu.VMEM((K, SLICE), jnp.float32) for _ in range(NBUF)] \
      + [pltpu.SemaphoreType.DMA for _ in range(2 * NBUF + 2)]

    def body(*refs):
        n_fixed = 5 if compute_deg else 4
        if compute_deg:
            (a_all, src_hbm, dst_hbm, o_all, deg_hbm) = refs[:5]
        else:
            (a_all, src_hbm, dst_hbm, o_all) = refs[:4]
        acc, sidx2, didx2 = refs[n_fixed:n_fixed + 3]
        bufs = refs[n_fixed + 3:n_fixed + 3 + NBUF]
        sems = refs[n_fixed + 3 + NBUF:]
        gsems = sems[:NBUF]
        ssems = sems[NBUF:2 * NBUF]
        isems = sems[2 * NBUF:2 * NBUF + 2]
        rows = bufs[0]

        c = lax.axis_index("c")
        s = lax.axis_index("s")
        base = s * RPT

        def fill_rows(val):
            def zb(i, carry):
                for q in range(SLICE // 16):
                    rows[i, pl.ds(q * 16, 16)] = jnp.full((16,), val,
                                                          jnp.float32)
                return carry
            lax.fori_loop(0, ZR, zb, 0)

        def zero_acc():
            fill_rows(0.0)
            for q in range(NZ):
                pltpu.sync_copy(rows, acc.at[pl.ds(base + q * ZR, ZR)])

        def write_acc(o_ref):
            pend = [None, None]
            for q in range(NZ):
                b = q % 2
                if pend[b] is not None:
                    pend[b].wait()
                pltpu.sync_copy(acc.at[pl.ds(base + q * ZR, ZR)], bufs[b])
                pend[b] = pltpu.async_copy(
                    bufs[b], o_ref.at[pl.ds(base + q * ZR, ZR)], gsems[b])
            for b in range(2):
                if pend[b] is not None:
                    pend[b].wait()

        def stage_idx(g, par, sync):
            dsts = (sidx2.at[par], didx2.at[par])
            srcs = (src_hbm, dst_hbm)
            if sync:
                for sr, dr in zip(srcs, dsts):
                    pltpu.sync_copy(sr.at[s].at[pl.ds(g * GB, GB)], dr)
                return None
            return [pltpu.async_copy(sr.at[s].at[pl.ds(g * GB, GB)], dr,
                                     isems[i])
                    for i, (sr, dr) in enumerate(zip(srcs, dsts))]

        def run_group(a_ref, sidx_g, didx_g):
            pend_g = [
                pltpu.async_copy(a_ref.at[sidx_g.at[tt]], bufs[tt], gsems[tt])
                for tt in range(NBUF)
            ]
            pend_s = [None] * NBUF
            for t in range(GB):
                b = t % NBUF
                if t >= 1 and (t - 1) + NBUF < GB:
                    bp = (t - 1) % NBUF
                    pend_s[bp].wait()
                    pend_s[bp] = None
                    pend_g[bp] = pltpu.async_copy(
                        a_ref.at[sidx_g.at[t - 1 + NBUF]], bufs[bp],
                        gsems[bp])
                pend_g[b].wait()
                pend_s[b] = pltpu.async_copy(
                    bufs[b], acc.at[didx_g.at[t]], ssems[b], add=True)
            for b in range(NBUF):
                if pend_s[b] is not None:
                    pend_s[b].wait()

        for p in range(n_passes):
            j = c * n_passes + p
            a_ref = a_all.at[j]
            o_ref = o_all.at[j]

            zero_acc()
            plsc.subcore_barrier()

            stage_idx(0, 0, True)

            def pair(u, carry):
                g0 = 2 * u
                pf = stage_idx(g0 + 1, 1, False)
                run_group(a_ref, sidx2.at[0], didx2.at[0])
                for d in pf:
                    d.wait()
                gnext = jnp.minimum(g0 + 2, NGRP - 2)
                pf2 = stage_idx(gnext, 0, False)
                run_group(a_ref, sidx2.at[1], didx2.at[1])
                for d in pf2:
                    d.wait()
                return carry
            lax.fori_loop(0, NHALF, pair, 0)
            plsc.subcore_barrier()

            write_acc(o_ref)
            plsc.subcore_barrier()

        if compute_deg:
            zero_acc()
            fill_rows(1.0)
            plsc.subcore_barrier()

            def dgroup(u, carry):
                g = c * NHALF + u
                stage_idx(g, 0, True)
                pend = [None] * NBUF
                for tt in range(GB):
                    b = tt % NBUF
                    if pend[b] is not None:
                        pend[b].wait()
                    pend[b] = pltpu.async_copy(
                        rows, acc.at[didx2.at[0].at[tt]], ssems[b], add=True)
                for b in range(NBUF):
                    if pend[b] is not None:
                        pend[b].wait()
                return carry
            lax.fori_loop(0, NHALF, dgroup, 0)
            plsc.subcore_barrier()

            write_acc(deg_hbm.at[c])

    return pl.kernel(body, out_type=tuple(out_type), mesh=_MESH,
                     scratch_types=scratch)


_seg2_deg = _make_seg_kernel(2, True)
_seg4 = _make_seg_kernel(4, False)


def _make_tc_layer(n_in, relu, sliced_out):
    d_in = n_in * SLICE
    RB = 1024
    grid = (NP // RB,)
    n_out = D_HID // SLICE

    def body(a_ref, h_ref, deg_ref, wl_ref, wr_ref, out_ref):
        accl = jnp.zeros((RB, D_HID), jnp.float32)
        accr = jnp.zeros((RB, D_HID), jnp.float32)
        for j in range(n_in):
            accl += jnp.dot(a_ref[j], wl_ref[j * SLICE:(j + 1) * SLICE, :],
                            preferred_element_type=jnp.float32)
            accr += jnp.dot(h_ref[j], wr_ref[j * SLICE:(j + 1) * SLICE, :],
                            preferred_element_type=jnp.float32)
        invd = 1.0 / jnp.maximum(deg_ref[0][:, 0:1] + deg_ref[1][:, 0:1],
                                 1.0)
        res = accl * invd + accr
        if relu:
            res = jnp.maximum(res, 0.0)
        if sliced_out:
            for j in range(n_out):
                out_ref[j] = res[:, j * SLICE:(j + 1) * SLICE]
        else:
            out_ref[...] = res

    in_specs = [
        pl.BlockSpec((n_in, RB, SLICE), lambda i: (0, i, 0)),
        pl.BlockSpec((n_in, RB, SLICE), lambda i: (0, i, 0)),
        pl.BlockSpec((NC, RB, SLICE), lambda i: (0, i, 0)),
        pl.BlockSpec((d_in, D_HID), lambda i: (0, 0)),
        pl.BlockSpec((d_in, D_HID), lambda i: (0, 0)),
    ]
    if sliced_out:
        out_specs = pl.BlockSpec((n_out, RB, SLICE), lambda i: (0, i, 0))
        out_shape = jax.ShapeDtypeStruct((n_out, NP, SLICE), jnp.float32)
    else:
        out_specs = pl.BlockSpec((RB, D_HID), lambda i: (i, 0))
        out_shape = jax.ShapeDtypeStruct((NP, D_HID), jnp.float32)

    return pl.pallas_call(body, grid=grid, in_specs=in_specs,
                          out_specs=out_specs, out_shape=out_shape)


_tc1 = _make_tc_layer(D_IN // SLICE, True, True)
_tc2 = _make_tc_layer(D_HID // SLICE, True, True)
_tc3 = _make_tc_layer(D_HID // SLICE, False, False)


def kernel(x, edge_index, W1_l, W1_r, W2_l, W2_r, W3_l, W3_r):
    e32 = edge_index.astype(jnp.int32)
    npad = EPAD - E
    src = jnp.concatenate([e32[0], jnp.zeros((npad,), jnp.int32)])
    dst = jnp.concatenate(
        [e32[1], N + (jnp.arange(npad, dtype=jnp.int32) % (NP - N))])
    src = src.reshape(NS, NCHUNK, K)
    dst = dst.reshape(NS, NCHUNK, K)

    x_all = jnp.pad(x.reshape(N, 2, SLICE).transpose(1, 0, 2),
                    ((0, 0), (0, NP - N), (0, 0)))

    a_all, deg = _seg2_deg(x_all, src, dst)
    h1 = _tc1(a_all, x_all, deg, W1_l, W1_r)

    (b_all,) = _seg4(h1, src, dst)
    h2 = _tc2(b_all, h1, deg, W2_l, W2_r)

    (c_all,) = _seg4(h2, src, dst)
    out = _tc3(c_all, h2, deg, W3_l, W3_r)
    return out[:N]
